# baseline (speedup 1.0000x reference)
import functools

import jax
import jax.numpy as jnp
from jax import lax
from jax.experimental import pallas as pl
from jax.experimental.pallas import tpu as pltpu
from jax.experimental.pallas import tpu_sc as plsc

B = 1024
F = 26
V = 100000
D = 16

NC = 2
NS = 16
L = 16
NW = NC * NS
BPW = B // NW
IPW = BPW * F
CH = 104
NCH = IPW // CH


def _afm_body(tab_hbm, idx_hbm, par_hbm, out_hbm, idxv, rows, tbuf, outv, parv, sem):
    wid = lax.axis_index("s") * NC + lax.axis_index("c")
    pltpu.sync_copy(idx_hbm.at[pl.ds(wid * NCH, NCH)], idxv)
    pltpu.sync_copy(par_hbm, parv)
    copies = [
        pltpu.async_copy(tab_hbm.at[idxv.at[j]], rows.at[pl.ds(j * CH, CH)], sem)
        for j in range(NCH)
    ]
    for c in copies:
        c.wait()

    lanes = lax.iota(jnp.int32, L)

    def per_sample(s, carry):
        acc = jnp.zeros((L,), jnp.float32)
        sq = jnp.zeros((L,), jnp.float32)
        rbase = s * F
        for i in range(F):
            v = rows[rbase + i, :]
            acc = acc + v
            sq = sq + v * v
        t = acc * acc - sq
        plsc.store_scatter(tbuf, [lanes * BPW + s], t)
        return carry

    lax.fori_loop(0, BPW, per_sample, 0)

    wo = parv[0, :]
    bo = parv[1, :]
    for g in range(BPW // L):
        tot = jnp.zeros((L,), jnp.float32)
        for d in range(L):
            tot = tot + tbuf[pl.ds(d * BPW + g * L, L)]
        z = (0.5 * tot) * wo + bo
        outv[pl.ds(g * L, L)] = 1.0 / (1.0 + jnp.exp(-z))

    pltpu.sync_copy(outv, out_hbm.at[pl.ds(wid * BPW, BPW)])


@jax.jit
def _afm_call(tab_flat, flat_idx, par):
    mesh = plsc.VectorSubcoreMesh(
        core_axis_name="c", subcore_axis_name="s", num_cores=NC, num_subcores=NS
    )
    run = functools.partial(
        pl.kernel,
        out_type=jax.ShapeDtypeStruct((B,), jnp.float32),
        mesh=mesh,
        compiler_params=pltpu.CompilerParams(
            needs_layout_passes=False, use_tc_tiling_on_sc=False
        ),
        scratch_types=[
            pltpu.VMEM((NCH, CH), jnp.int32),
            pltpu.VMEM((IPW, D), jnp.float32),
            pltpu.VMEM((L * BPW,), jnp.float32),
            pltpu.VMEM((BPW,), jnp.float32),
            pltpu.VMEM((2, L), jnp.float32),
            pltpu.SemaphoreType.DMA,
        ],
    )(_afm_body)
    return run(tab_flat, flat_idx, par)


def kernel(inputs, tables, W1, b1, W2, b2, Wo, bo):
    sparse = inputs[:, 13:]
    flat_idx = (sparse + jnp.arange(F, dtype=jnp.int32)[None, :] * V).reshape(
        NW * NCH, CH
    )
    tab_flat = tables.reshape(F * V, D)
    par = jnp.stack(
        [jnp.full((L,), Wo[0, 0], jnp.float32), jnp.full((L,), bo[0], jnp.float32)]
    )
    out = _afm_call(tab_flat, flat_idx, par)
    return out.reshape(B, 1)


# speedup vs baseline: 3.8209x; 3.8209x over previous
import functools

import jax
import jax.numpy as jnp
from jax import lax
from jax.experimental import pallas as pl
from jax.experimental.pallas import tpu as pltpu
from jax.experimental.pallas import tpu_sc as plsc

B = 1024
F = 26
V = 100000
D = 16

NC = 2
NS = 16
L = 16
NW = NC * NS
BPW = B // NW
IPW = BPW * F
EPW = IPW * D
ECH = 128
NECH = EPW // ECH


def _afm_body(tab_hbm, idx_hbm, par_hbm, out_hbm, idxv, idxg, rows, tbuf, outv, parv, sem):
    wid = lax.axis_index("s") * NC + lax.axis_index("c")
    pltpu.sync_copy(idx_hbm.at[pl.ds(wid * IPW, IPW)], idxv)
    pltpu.sync_copy(par_hbm, parv)
    dstride = lax.iota(jnp.int32, L) * V
    for q in range(IPW // L):
        pbv = idxv[pl.ds(q * L, L)]
        for l in range(L):
            idxg[pl.ds((q * L + l) * D, D)] = pbv[l] + dstride
    copies = [
        pltpu.async_copy(
            tab_hbm.at[idxg.at[pl.ds(j * ECH, ECH)]], rows.at[pl.ds(j * ECH, ECH)], sem
        )
        for j in range(NECH)
    ]
    for c in copies:
        c.wait()

    lanes = lax.iota(jnp.int32, L)

    def per_sample(s, carry):
        acc = jnp.zeros((L,), jnp.float32)
        sq = jnp.zeros((L,), jnp.float32)
        pbase = s * F
        for i in range(F):
            v = rows[pl.ds((pbase + i) * D, D)]
            acc = acc + v
            sq = sq + v * v
        t = acc * acc - sq
        plsc.store_scatter(tbuf, [lanes * BPW + s], t)
        return carry

    lax.fori_loop(0, BPW, per_sample, 0)

    wo = parv[0, :]
    bo = parv[1, :]
    for g in range(BPW // L):
        tot = jnp.zeros((L,), jnp.float32)
        for d in range(L):
            tot = tot + tbuf[pl.ds(d * BPW + g * L, L)]
        z = (0.5 * tot) * wo + bo
        outv[pl.ds(g * L, L)] = 1.0 / (1.0 + jnp.exp(-z))

    pltpu.sync_copy(outv, out_hbm.at[pl.ds(wid * BPW, BPW)])


@jax.jit
def _afm_call(tab_flat, base_idx, par):
    mesh = plsc.VectorSubcoreMesh(
        core_axis_name="c", subcore_axis_name="s", num_cores=NC, num_subcores=NS
    )
    run = functools.partial(
        pl.kernel,
        out_type=jax.ShapeDtypeStruct((B,), jnp.float32),
        mesh=mesh,
        compiler_params=pltpu.CompilerParams(
            needs_layout_passes=False, use_tc_tiling_on_sc=False
        ),
        scratch_types=[
            pltpu.VMEM((IPW,), jnp.int32),
            pltpu.VMEM((EPW,), jnp.int32),
            pltpu.VMEM((EPW,), jnp.float32),
            pltpu.VMEM((L * BPW,), jnp.float32),
            pltpu.VMEM((BPW,), jnp.float32),
            pltpu.VMEM((2, L), jnp.float32),
            pltpu.SemaphoreType.DMA,
        ],
    )(_afm_body)
    return run(tab_flat, base_idx, par)


def kernel(inputs, tables, W1, b1, W2, b2, Wo, bo):
    sparse = inputs[:, 13:]
    base_idx = (sparse + jnp.arange(F, dtype=jnp.int32)[None, :] * (D * V)).reshape(
        B * F
    )
    tab_flat = tables.transpose(0, 2, 1).reshape(F * D * V)
    par = jnp.stack(
        [jnp.full((L,), Wo[0, 0], jnp.float32), jnp.full((L,), bo[0], jnp.float32)]
    )
    out = _afm_call(tab_flat, base_idx, par)
    return out.reshape(B, 1)


# speedup vs baseline: 10.2107x; 2.6723x over previous
import functools

import jax
import jax.numpy as jnp
from jax import lax
from jax.experimental import pallas as pl
from jax.experimental.pallas import tpu as pltpu
from jax.experimental.pallas import tpu_sc as plsc

B = 1024
F = 26
V = 100000
D = 16

NC = 2
NS = 16
L = 16
FPC = F // NC
NW = NC * NS
BPW = B // NW
NQ = B // L


def _sweep_body(tab_hbm, idx_hbm, out_hbm, rowbuf, idxv, accsq, sem):
    c = lax.axis_index("c")
    t = lax.axis_index("s")
    pltpu.sync_copy(idx_hbm.at[pl.ds(c * L, L), :], idxv)
    zeros = jnp.zeros((L,), jnp.float32)
    for q in range(NQ):
        accsq[0, pl.ds(q * L, L)] = zeros
        accsq[1, pl.ds(q * L, L)] = zeros

    def per_field(i, chk):
        dep = lax.shift_right_logical(
            lax.convert_element_type(jnp.abs(chk[0]), jnp.int32), 31
        )
        gf = c * FPC + i
        r = gf * D + t + dep
        pltpu.async_copy(tab_hbm.at[r // 8, r % 8, :], rowbuf, sem).wait()
        for q in range(NQ):
            vv = idxv[i, pl.ds(q * L, L)]
            val = plsc.load_gather(rowbuf, [vv])
            chk = chk + val
            a = accsq[0, pl.ds(q * L, L)]
            accsq[0, pl.ds(q * L, L)] = a + val
            s = accsq[1, pl.ds(q * L, L)]
            accsq[1, pl.ds(q * L, L)] = s + val * val
        return chk

    lax.fori_loop(0, FPC, per_field, jnp.zeros((L,), jnp.float32))
    pltpu.sync_copy(accsq, out_hbm.at[c, t])


def _head_body(part_hbm, par_hbm, out_hbm, pbuf, parv, outv):
    wid = lax.axis_index("s") * NC + lax.axis_index("c")
    base = wid * BPW
    pltpu.sync_copy(par_hbm, parv)
    for c in range(NC):
        for k in range(2):
            pltpu.sync_copy(
                part_hbm.at[c, :, pl.ds(k * B + base, BPW)], pbuf.at[c * 2 + k]
            )
    wo = parv[0, :]
    bo = parv[1, :]
    for g in range(BPW // L):
        tot = jnp.zeros((L,), jnp.float32)
        for d in range(D):
            a = pbuf[0, d, pl.ds(g * L, L)] + pbuf[2, d, pl.ds(g * L, L)]
            q = pbuf[1, d, pl.ds(g * L, L)] + pbuf[3, d, pl.ds(g * L, L)]
            tot = tot + (a * a - q)
        z = (0.5 * tot) * wo + bo
        outv[pl.ds(g * L, L)] = 1.0 / (1.0 + jnp.exp(-z))
    pltpu.sync_copy(outv, out_hbm.at[pl.ds(base, BPW)])


@jax.jit
def _afm_call(tab3, idx32, par):
    mesh = plsc.VectorSubcoreMesh(
        core_axis_name="c", subcore_axis_name="s", num_cores=NC, num_subcores=NS
    )
    sweep = functools.partial(
        pl.kernel,
        out_type=jax.ShapeDtypeStruct((NC, D, 8, B), jnp.float32),
        mesh=mesh,
        compiler_params=pltpu.CompilerParams(needs_layout_passes=False),
        scratch_types=[
            pltpu.VMEM((V,), jnp.float32),
            pltpu.VMEM((L, B), jnp.int32),
            pltpu.VMEM((8, B), jnp.float32),
            pltpu.SemaphoreType.DMA,
        ],
    )(_sweep_body)
    partials = sweep(tab3, idx32)

    head = functools.partial(
        pl.kernel,
        out_type=jax.ShapeDtypeStruct((B,), jnp.float32),
        mesh=mesh,
        compiler_params=pltpu.CompilerParams(
            needs_layout_passes=False, use_tc_tiling_on_sc=False
        ),
        scratch_types=[
            pltpu.VMEM((4, D, BPW), jnp.float32),
            pltpu.VMEM((2, L), jnp.float32),
            pltpu.VMEM((BPW,), jnp.float32),
        ],
    )(_head_body)
    out = head(partials.reshape(NC, D, 8 * B), par)
    return out


def kernel(inputs, tables, W1, b1, W2, b2, Wo, bo):
    sparse = inputs[:, 13:]
    spT = sparse.T
    pad = jnp.zeros((NC * L - F, B), jnp.int32)
    idx32 = jnp.concatenate(
        [spT[:FPC], pad[: L - FPC], spT[FPC:], pad[L - FPC :]], axis=0
    )
    tab3 = tables.transpose(0, 2, 1).reshape(F * D // 8, 8, V)
    par = jnp.stack(
        [jnp.full((L,), Wo[0, 0], jnp.float32), jnp.full((L,), bo[0], jnp.float32)]
    )
    out = _afm_call(tab3, idx32, par)
    return out.reshape(B, 1)
